# Initial kernel scaffold; baseline (speedup 1.0000x reference)
import functools

import jax
import jax.numpy as jnp
from jax.experimental import pallas as pl
from jax.experimental.pallas import tpu as pltpu

_K = 10


def _proj_body(x_ref, w_ref, e_ref):
    x = x_ref[...]
    xa = jnp.mean(x, axis=1)
    w = w_ref[...]
    e = jax.lax.dot_general(
        xa, w, (((1,), (1,)), ((), ())),
        preferred_element_type=jnp.float32,
        precision=jax.lax.Precision.HIGHEST)
    n = jnp.sqrt(jnp.sum(e * e, axis=1, keepdims=True))
    e_ref[...] = e / jnp.maximum(n, 1e-12)


def _topk_body(alpha_ref, e_rows_ref, e_all_ref, vals_ref, idx_ref, *, rb, n, k):
    r = pl.program_id(0)
    e_rows = e_rows_ref[...]
    e_all = e_all_ref[...]
    sim = jax.lax.dot_general(
        e_rows, e_all, (((1,), (1,)), ((), ())),
        preferred_element_type=jnp.float32,
        precision=jax.lax.Precision.HIGHEST)
    col = jax.lax.broadcasted_iota(jnp.int32, (rb, n), 1)
    row_g = r * rb + jax.lax.broadcasted_iota(jnp.int32, (rb, n), 0)
    sim = jnp.where(col == row_g, -2.0, sim)
    alpha = alpha_ref[0]
    vals_list, idx_list = [], []
    for _ in range(k):
        m = jnp.max(sim, axis=1, keepdims=True)
        hit = sim == m
        imax = jnp.min(jnp.where(hit, col, n), axis=1, keepdims=True)
        vals_list.append(m[:, 0] * alpha)
        idx_list.append(imax[:, 0])
        sim = jnp.where(col == imax, -2.0, sim)
    vals_ref[...] = jnp.stack(vals_list, axis=1)
    idx_ref[...] = jnp.stack(idx_list, axis=1)


def _scale_body(alpha_ref, a_ref, o_ref):
    o_ref[...] = a_ref[...] * (1.0 - alpha_ref[0])


def kernel(x, fixed_edge_index, fixed_edge_attr, W_proj, mix_logit):
    n, t, h = x.shape
    hp = W_proj.shape[0]
    e_num = fixed_edge_attr.shape[0]
    idx_dtype = fixed_edge_index.dtype

    alpha = jax.nn.sigmoid(mix_logit).astype(jnp.float32).reshape(1)

    r1 = 1000
    e = pl.pallas_call(
        _proj_body,
        grid=(n // r1,),
        in_specs=[
            pl.BlockSpec((r1, t, h), lambda i: (i, 0, 0)),
            pl.BlockSpec((hp, h), lambda i: (0, 0)),
        ],
        out_specs=pl.BlockSpec((r1, hp), lambda i: (i, 0)),
        out_shape=jax.ShapeDtypeStruct((n, hp), jnp.float32),
    )(x, W_proj)

    rb = 400
    vals, idx = pl.pallas_call(
        functools.partial(_topk_body, rb=rb, n=n, k=_K),
        grid=(n // rb,),
        in_specs=[
            pl.BlockSpec(memory_space=pltpu.SMEM),
            pl.BlockSpec((rb, hp), lambda i: (i, 0)),
            pl.BlockSpec((n, hp), lambda i: (0, 0)),
        ],
        out_specs=[
            pl.BlockSpec((rb, _K), lambda i: (i, 0)),
            pl.BlockSpec((rb, _K), lambda i: (i, 0)),
        ],
        out_shape=[
            jax.ShapeDtypeStruct((n, _K), jnp.float32),
            jax.ShapeDtypeStruct((n, _K), jnp.int32),
        ],
    )(alpha, e, e)

    fa = fixed_edge_attr.reshape(e_num // 128, 128)
    fa_scaled = pl.pallas_call(
        _scale_body,
        in_specs=[
            pl.BlockSpec(memory_space=pltpu.SMEM),
            pl.BlockSpec(fa.shape, lambda: (0, 0)),
        ],
        out_specs=pl.BlockSpec(fa.shape, lambda: (0, 0)),
        out_shape=jax.ShapeDtypeStruct(fa.shape, jnp.float32),
    )(alpha, fa)

    src = jnp.repeat(jnp.arange(n, dtype=idx_dtype), _K)
    dst = idx.reshape(-1).astype(idx_dtype)
    dyn_edge_index = jnp.stack([src, dst], axis=0)
    combined_edge_index = jnp.concatenate([fixed_edge_index, dyn_edge_index], axis=1)
    combined_edge_attr = jnp.concatenate(
        [fa_scaled.reshape(e_num, 1), vals.reshape(-1, 1)], axis=0)
    return combined_edge_index, combined_edge_attr


# speedup vs baseline: 40.0326x; 40.0326x over previous
import functools

import jax
import jax.numpy as jnp
from jax.experimental import pallas as pl
from jax.experimental.pallas import tpu as pltpu

_K = 10
_LANES = 128


def _proj_body(x_ref, w_ref, e_ref):
    x = x_ref[...]
    xa = jnp.mean(x, axis=1)
    w = w_ref[...]
    e = jax.lax.dot_general(
        xa, w, (((1,), (1,)), ((), ())),
        preferred_element_type=jnp.float32,
        precision=jax.lax.Precision.HIGHEST)
    nrm = jnp.sqrt(jnp.sum(e * e, axis=1, keepdims=True))
    e_ref[...] = e / jnp.maximum(nrm, 1e-12)


def _topk_body(alpha_ref, e_rows_ref, e_all_ref, vals_ref, idx_ref, sim_ref,
               *, rb, n, k):
    r = pl.program_id(0)
    e_rows = e_rows_ref[...]
    e_all = e_all_ref[...]
    sim = jax.lax.dot_general(
        e_rows, e_all, (((1,), (1,)), ((), ())),
        preferred_element_type=jnp.float32,
        precision=jax.lax.Precision.HIGHEST)
    col = jax.lax.broadcasted_iota(jnp.int32, (rb, n), 1)
    row_g = r * rb + jax.lax.broadcasted_iota(jnp.int32, (rb, n), 0)
    sim_ref[...] = jnp.where(col == row_g, -2.0, sim)

    lane = jax.lax.broadcasted_iota(jnp.int32, (rb, _LANES), 1)

    def it(j, carry):
        vals_c, idx_c = carry
        s = sim_ref[...]
        m = jnp.max(s, axis=1, keepdims=True)
        imax = jnp.min(jnp.where(s == m, col, n), axis=1, keepdims=True)
        sim_ref[...] = jnp.where(col == imax, -2.0, s)
        vals_c = jnp.where(lane == j, m, vals_c)
        idx_c = jnp.where(lane == j, imax, idx_c)
        return vals_c, idx_c

    vals0 = jnp.zeros((rb, _LANES), jnp.float32)
    idx0 = jnp.zeros((rb, _LANES), jnp.int32)
    vals_c, idx_c = jax.lax.fori_loop(
        jnp.int32(0), jnp.int32(k), it, (vals0, idx0))
    vals_ref[...] = vals_c * alpha_ref[0]
    idx_ref[...] = idx_c


def _scale_body(alpha_ref, a_ref, o_ref):
    o_ref[...] = a_ref[...] * (1.0 - alpha_ref[0])


def kernel(x, fixed_edge_index, fixed_edge_attr, W_proj, mix_logit):
    n, t, h = x.shape
    hp = W_proj.shape[0]
    e_num = fixed_edge_attr.shape[0]
    idx_dtype = fixed_edge_index.dtype
    attr_dtype = jnp.promote_types(fixed_edge_attr.dtype, W_proj.dtype)
    w32 = W_proj.astype(jnp.float32)

    alpha = jax.nn.sigmoid(mix_logit).astype(jnp.float32).reshape(1)

    r1 = 1000
    e = pl.pallas_call(
        _proj_body,
        grid=(n // r1,),
        in_specs=[
            pl.BlockSpec((r1, t, h), lambda i: (i, i * 0, i * 0)),
            pl.BlockSpec((hp, h), lambda i: (i * 0, i * 0)),
        ],
        out_specs=pl.BlockSpec((r1, hp), lambda i: (i, i * 0)),
        out_shape=jax.ShapeDtypeStruct((n, hp), jnp.float32),
    )(x, w32)

    rb = 80
    vals, idx = pl.pallas_call(
        functools.partial(_topk_body, rb=rb, n=n, k=_K),
        grid=(n // rb,),
        in_specs=[
            pl.BlockSpec((1,), lambda i: (i * 0,), memory_space=pltpu.SMEM),
            pl.BlockSpec((rb, hp), lambda i: (i, i * 0)),
            pl.BlockSpec((n, hp), lambda i: (i * 0, i * 0)),
        ],
        out_specs=[
            pl.BlockSpec((rb, _LANES), lambda i: (i, i * 0)),
            pl.BlockSpec((rb, _LANES), lambda i: (i, i * 0)),
        ],
        out_shape=[
            jax.ShapeDtypeStruct((n, _LANES), jnp.float32),
            jax.ShapeDtypeStruct((n, _LANES), jnp.int32),
        ],
        scratch_shapes=[pltpu.VMEM((rb, n), jnp.float32)],
    )(alpha, e, e)

    fa = fixed_edge_attr.reshape(e_num // _LANES, _LANES)
    fa_scaled = pl.pallas_call(
        _scale_body,
        grid=(1,),
        in_specs=[
            pl.BlockSpec((1,), lambda i: (i * 0,), memory_space=pltpu.SMEM),
            pl.BlockSpec(fa.shape, lambda i: (i * 0, i * 0)),
        ],
        out_specs=pl.BlockSpec(fa.shape, lambda i: (i * 0, i * 0)),
        out_shape=jax.ShapeDtypeStruct(fa.shape, jnp.float32),
    )(alpha, fa)

    src = jnp.repeat(jnp.arange(n, dtype=idx_dtype), _K)
    dst = idx[:, :_K].reshape(-1).astype(idx_dtype)
    dyn_edge_index = jnp.stack([src, dst], axis=0)
    combined_edge_index = jnp.concatenate([fixed_edge_index, dyn_edge_index], axis=1)
    combined_edge_attr = jnp.concatenate(
        [fa_scaled.reshape(e_num, 1), vals[:, :_K].reshape(-1, 1)], axis=0
    ).astype(attr_dtype)
    return combined_edge_index, combined_edge_attr


# speedup vs baseline: 54.6135x; 1.3642x over previous
import functools

import jax
import jax.numpy as jnp
from jax.experimental import pallas as pl
from jax.experimental.pallas import tpu as pltpu

_K = 10
_LANES = 128


def _proj_body(x_ref, w_ref, e_ref):
    x = x_ref[...]
    xa = jnp.mean(x, axis=1)
    w = w_ref[...]
    e = jax.lax.dot_general(
        xa, w, (((1,), (1,)), ((), ())),
        preferred_element_type=jnp.float32,
        precision=jax.lax.Precision.HIGHEST)
    nrm = jnp.sqrt(jnp.sum(e * e, axis=1, keepdims=True))
    e_ref[...] = e / jnp.maximum(nrm, 1e-12)


def _topk_body(alpha_ref, e_rows_ref, e_all_ref, vals_ref, idx_ref, sim_ref,
               *, rb, n, k):
    r = pl.program_id(0)
    e_rows = e_rows_ref[...]
    e_all = e_all_ref[...]
    sim = jax.lax.dot_general(
        e_rows, e_all, (((1,), (1,)), ((), ())),
        preferred_element_type=jnp.float32,
        precision=jax.lax.Precision.HIGHEST)
    col = jax.lax.broadcasted_iota(jnp.int32, (rb, n), 1)
    row_g = r * rb + jax.lax.broadcasted_iota(jnp.int32, (rb, n), 0)
    sim_ref[...] = jnp.where(col == row_g, -2.0, sim)

    lane = jax.lax.broadcasted_iota(jnp.int32, (rb, _LANES), 1)

    def it(j, carry):
        vals_c, idx_c = carry
        j2 = j * 2
        s = sim_ref[...]
        m = jnp.max(s, axis=1, keepdims=True)
        imax = jnp.min(jnp.where(s == m, col, n), axis=1, keepdims=True)
        s = jnp.where(col == imax, -2.0, s)
        vals_c = jnp.where(lane == j2, m, vals_c)
        idx_c = jnp.where(lane == j2, imax, idx_c)
        m = jnp.max(s, axis=1, keepdims=True)
        imax2 = jnp.min(jnp.where(s == m, col, n), axis=1, keepdims=True)
        sim_ref[...] = jnp.where(col == imax2, -2.0, s)
        vals_c = jnp.where(lane == j2 + 1, m, vals_c)
        idx_c = jnp.where(lane == j2 + 1, imax2, idx_c)
        return vals_c, idx_c

    vals0 = jnp.zeros((rb, _LANES), jnp.float32)
    idx0 = jnp.zeros((rb, _LANES), jnp.int32)
    vals_c, idx_c = jax.lax.fori_loop(
        jnp.int32(0), jnp.int32(k // 2), it, (vals0, idx0))
    vals_ref[...] = vals_c * alpha_ref[0]
    idx_ref[...] = idx_c


def _scale_body(alpha_ref, a_ref, o_ref):
    o_ref[...] = a_ref[...] * (1.0 - alpha_ref[0])


def kernel(x, fixed_edge_index, fixed_edge_attr, W_proj, mix_logit):
    n, t, h = x.shape
    hp = W_proj.shape[0]
    e_num = fixed_edge_attr.shape[0]
    idx_dtype = fixed_edge_index.dtype
    attr_dtype = jnp.promote_types(fixed_edge_attr.dtype, W_proj.dtype)
    w32 = W_proj.astype(jnp.float32)

    alpha = jax.nn.sigmoid(mix_logit).astype(jnp.float32).reshape(1)

    r1 = 1000
    e = pl.pallas_call(
        _proj_body,
        grid=(n // r1,),
        in_specs=[
            pl.BlockSpec((r1, t, h), lambda i: (i, i * 0, i * 0)),
            pl.BlockSpec((hp, h), lambda i: (i * 0, i * 0)),
        ],
        out_specs=pl.BlockSpec((r1, hp), lambda i: (i, i * 0)),
        out_shape=jax.ShapeDtypeStruct((n, hp), jnp.float32),
    )(x, w32)

    rb = 200
    vals, idx = pl.pallas_call(
        functools.partial(_topk_body, rb=rb, n=n, k=_K),
        grid=(n // rb,),
        in_specs=[
            pl.BlockSpec((1,), lambda i: (i * 0,), memory_space=pltpu.SMEM),
            pl.BlockSpec((rb, hp), lambda i: (i, i * 0)),
            pl.BlockSpec((n, hp), lambda i: (i * 0, i * 0)),
        ],
        out_specs=[
            pl.BlockSpec((rb, _LANES), lambda i: (i, i * 0)),
            pl.BlockSpec((rb, _LANES), lambda i: (i, i * 0)),
        ],
        out_shape=[
            jax.ShapeDtypeStruct((n, _LANES), jnp.float32),
            jax.ShapeDtypeStruct((n, _LANES), jnp.int32),
        ],
        scratch_shapes=[pltpu.VMEM((rb, n), jnp.float32)],
    )(alpha, e, e)

    fa = fixed_edge_attr.reshape(e_num // _LANES, _LANES)
    fa_scaled = pl.pallas_call(
        _scale_body,
        grid=(1,),
        in_specs=[
            pl.BlockSpec((1,), lambda i: (i * 0,), memory_space=pltpu.SMEM),
            pl.BlockSpec(fa.shape, lambda i: (i * 0, i * 0)),
        ],
        out_specs=pl.BlockSpec(fa.shape, lambda i: (i * 0, i * 0)),
        out_shape=jax.ShapeDtypeStruct(fa.shape, jnp.float32),
    )(alpha, fa)

    src = jnp.repeat(jnp.arange(n, dtype=idx_dtype), _K)
    dst = idx[:, :_K].reshape(-1).astype(idx_dtype)
    dyn_edge_index = jnp.stack([src, dst], axis=0)
    combined_edge_index = jnp.concatenate([fixed_edge_index, dyn_edge_index], axis=1)
    combined_edge_attr = jnp.concatenate(
        [fa_scaled.reshape(e_num, 1), vals[:, :_K].reshape(-1, 1)], axis=0
    ).astype(attr_dtype)
    return combined_edge_index, combined_edge_attr


# speedup vs baseline: 55.0198x; 1.0074x over previous
import functools

import jax
import jax.numpy as jnp
from jax.experimental import pallas as pl
from jax.experimental.pallas import tpu as pltpu

_K = 10
_LANES = 128


def _proj_body(x_ref, w_ref, e_ref):
    x = x_ref[...]
    xa = jnp.mean(x, axis=1)
    w = w_ref[...]
    e = jax.lax.dot_general(
        xa, w, (((1,), (1,)), ((), ())),
        preferred_element_type=jnp.float32,
        precision=jax.lax.Precision.HIGHEST)
    nrm = jnp.sqrt(jnp.sum(e * e, axis=1, keepdims=True))
    e_ref[...] = e / jnp.maximum(nrm, 1e-12)


def _topk_body(alpha_ref, e_rows_ref, e_all_ref, vals_ref, idx_ref, sim_ref,
               *, rb, n, k, g):
    i = pl.program_id(0)
    par = jax.lax.rem(i, jnp.int32(2))
    col = jax.lax.broadcasted_iota(jnp.int32, (rb, n), 1)

    @pl.when(i < g)
    def _fill():
        e_rows = e_rows_ref[...]
        e_all = e_all_ref[...]
        sim = jax.lax.dot_general(
            e_rows, e_all, (((1,), (1,)), ((), ())),
            preferred_element_type=jnp.float32,
            precision=jax.lax.Precision.HIGHEST)
        row_g = i * rb + jax.lax.broadcasted_iota(jnp.int32, (rb, n), 0)
        sim_ref[par] = jnp.where(col == row_g, -2.0, sim)

    lane = jax.lax.broadcasted_iota(jnp.int32, (rb, _LANES), 1)
    other = 1 - par

    def it(j, carry):
        vals_c, idx_c = carry
        j2 = j * 2
        s = sim_ref[other]
        m = jnp.max(s, axis=1, keepdims=True)
        imax = jnp.min(jnp.where(s == m, col, n), axis=1, keepdims=True)
        s = jnp.where(col == imax, -2.0, s)
        vals_c = jnp.where(lane == j2, m, vals_c)
        idx_c = jnp.where(lane == j2, imax, idx_c)
        m = jnp.max(s, axis=1, keepdims=True)
        imax2 = jnp.min(jnp.where(s == m, col, n), axis=1, keepdims=True)
        sim_ref[other] = jnp.where(col == imax2, -2.0, s)
        vals_c = jnp.where(lane == j2 + 1, m, vals_c)
        idx_c = jnp.where(lane == j2 + 1, imax2, idx_c)
        return vals_c, idx_c

    @pl.when(i > 0)
    def _drain():
        vals0 = jnp.zeros((rb, _LANES), jnp.float32)
        idx0 = jnp.zeros((rb, _LANES), jnp.int32)
        vals_c, idx_c = jax.lax.fori_loop(
            jnp.int32(0), jnp.int32(k // 2), it, (vals0, idx0))
        vals_ref[...] = vals_c * alpha_ref[0]
        idx_ref[...] = idx_c


def _scale_body(alpha_ref, a_ref, o_ref):
    o_ref[...] = a_ref[...] * (1.0 - alpha_ref[0])


def kernel(x, fixed_edge_index, fixed_edge_attr, W_proj, mix_logit):
    n, t, h = x.shape
    hp = W_proj.shape[0]
    e_num = fixed_edge_attr.shape[0]
    idx_dtype = fixed_edge_index.dtype
    attr_dtype = jnp.promote_types(fixed_edge_attr.dtype, W_proj.dtype)
    w32 = W_proj.astype(jnp.float32)

    alpha = jax.nn.sigmoid(mix_logit).astype(jnp.float32).reshape(1)

    r1 = 1000
    e = pl.pallas_call(
        _proj_body,
        grid=(n // r1,),
        in_specs=[
            pl.BlockSpec((r1, t, h), lambda i: (i, i * 0, i * 0)),
            pl.BlockSpec((hp, h), lambda i: (i * 0, i * 0)),
        ],
        out_specs=pl.BlockSpec((r1, hp), lambda i: (i, i * 0)),
        out_shape=jax.ShapeDtypeStruct((n, hp), jnp.float32),
    )(x, w32)

    rb = 200
    g = n // rb
    vals, idx = pl.pallas_call(
        functools.partial(_topk_body, rb=rb, n=n, k=_K, g=g),
        grid=(g + 1,),
        in_specs=[
            pl.BlockSpec((1,), lambda i: (i * 0,), memory_space=pltpu.SMEM),
            pl.BlockSpec((rb, hp), lambda i: (jnp.minimum(i, g - 1), i * 0)),
            pl.BlockSpec((n, hp), lambda i: (i * 0, i * 0)),
        ],
        out_specs=[
            pl.BlockSpec((rb, _LANES), lambda i: (jnp.maximum(i - 1, 0), i * 0)),
            pl.BlockSpec((rb, _LANES), lambda i: (jnp.maximum(i - 1, 0), i * 0)),
        ],
        out_shape=[
            jax.ShapeDtypeStruct((n, _LANES), jnp.float32),
            jax.ShapeDtypeStruct((n, _LANES), jnp.int32),
        ],
        scratch_shapes=[pltpu.VMEM((2, rb, n), jnp.float32)],
    )(alpha, e, e)

    fa = fixed_edge_attr.reshape(e_num // _LANES, _LANES)
    fa_scaled = pl.pallas_call(
        _scale_body,
        grid=(1,),
        in_specs=[
            pl.BlockSpec((1,), lambda i: (i * 0,), memory_space=pltpu.SMEM),
            pl.BlockSpec(fa.shape, lambda i: (i * 0, i * 0)),
        ],
        out_specs=pl.BlockSpec(fa.shape, lambda i: (i * 0, i * 0)),
        out_shape=jax.ShapeDtypeStruct(fa.shape, jnp.float32),
    )(alpha, fa)

    src = jnp.repeat(jnp.arange(n, dtype=idx_dtype), _K)
    dst = idx[:, :_K].reshape(-1).astype(idx_dtype)
    dyn_edge_index = jnp.stack([src, dst], axis=0)
    combined_edge_index = jnp.concatenate([fixed_edge_index, dyn_edge_index], axis=1)
    combined_edge_attr = jnp.concatenate(
        [fa_scaled.reshape(e_num, 1), vals[:, :_K].reshape(-1, 1)], axis=0
    ).astype(attr_dtype)
    return combined_edge_index, combined_edge_attr


# speedup vs baseline: 56.8884x; 1.0340x over previous
import functools

import jax
import jax.numpy as jnp
from jax.experimental import pallas as pl
from jax.experimental.pallas import tpu as pltpu

_K = 10
_LANES = 128


def _proj_body(x_ref, w_ref, e_ref):
    x = x_ref[...]
    xa = jnp.mean(x, axis=1)
    w = w_ref[...]
    e = jax.lax.dot_general(
        xa, w, (((1,), (1,)), ((), ())),
        preferred_element_type=jnp.float32,
        precision=jax.lax.Precision.HIGHEST)
    nrm = jnp.sqrt(jnp.sum(e * e, axis=1, keepdims=True))
    e_ref[...] = e / jnp.maximum(nrm, 1e-12)


def _topk_body(alpha_ref, e_rows_ref, e_all_ref, vals_ref, idx_ref, sim_ref,
               *, rb, n, k, g):
    i = pl.program_id(0)
    par = jax.lax.rem(i, jnp.int32(2))
    col = jax.lax.broadcasted_iota(jnp.int32, (rb, n), 1)

    @pl.when(i < g)
    def _fill():
        e_rows = e_rows_ref[...]
        e_all = e_all_ref[...]
        sim = jax.lax.dot_general(
            e_rows, e_all, (((1,), (1,)), ((), ())),
            preferred_element_type=jnp.float32,
            precision=jax.lax.Precision.HIGHEST)
        row_g = i * rb + jax.lax.broadcasted_iota(jnp.int32, (rb, n), 0)
        sim_ref[par] = jnp.where(col == row_g, -2.0, sim)

    lane = jax.lax.broadcasted_iota(jnp.int32, (rb, _LANES), 1)
    other = 1 - par

    def it(j, carry):
        vals_c, idx_c = carry
        j2 = j * 2
        s = sim_ref[other]
        m = jnp.max(s, axis=1, keepdims=True)
        imax = jnp.min(jnp.where(s == m, col, n), axis=1, keepdims=True)
        s = jnp.where(col == imax, -2.0, s)
        vals_c = jnp.where(lane == j2, m, vals_c)
        idx_c = jnp.where(lane == j2, imax, idx_c)
        m = jnp.max(s, axis=1, keepdims=True)
        imax2 = jnp.min(jnp.where(s == m, col, n), axis=1, keepdims=True)
        sim_ref[other] = jnp.where(col == imax2, -2.0, s)
        vals_c = jnp.where(lane == j2 + 1, m, vals_c)
        idx_c = jnp.where(lane == j2 + 1, imax2, idx_c)
        return vals_c, idx_c

    @pl.when(i > 0)
    def _drain():
        vals0 = jnp.zeros((rb, _LANES), jnp.float32)
        idx0 = jnp.zeros((rb, _LANES), jnp.int32)
        vals_c, idx_c = jax.lax.fori_loop(
            jnp.int32(0), jnp.int32(k // 2), it, (vals0, idx0))
        vals_ref[...] = vals_c * alpha_ref[0]
        idx_ref[...] = idx_c


def _scale_body(alpha_ref, a_ref, o_ref):
    o_ref[...] = a_ref[...] * (1.0 - alpha_ref[0])


def kernel(x, fixed_edge_index, fixed_edge_attr, W_proj, mix_logit):
    n, t, h = x.shape
    hp = W_proj.shape[0]
    e_num = fixed_edge_attr.shape[0]
    idx_dtype = fixed_edge_index.dtype
    attr_dtype = jnp.promote_types(fixed_edge_attr.dtype, W_proj.dtype)
    w32 = W_proj.astype(jnp.float32)

    alpha = jax.nn.sigmoid(mix_logit).astype(jnp.float32).reshape(1)

    r1 = 1000
    e = pl.pallas_call(
        _proj_body,
        grid=(n // r1,),
        in_specs=[
            pl.BlockSpec((r1, t, h), lambda i: (i, i * 0, i * 0)),
            pl.BlockSpec((hp, h), lambda i: (i * 0, i * 0)),
        ],
        out_specs=pl.BlockSpec((r1, hp), lambda i: (i, i * 0)),
        out_shape=jax.ShapeDtypeStruct((n, hp), jnp.float32),
    )(x, w32)

    rb = 400
    g = n // rb
    vals, idx = pl.pallas_call(
        functools.partial(_topk_body, rb=rb, n=n, k=_K, g=g),
        grid=(g + 1,),
        in_specs=[
            pl.BlockSpec((1,), lambda i: (i * 0,), memory_space=pltpu.SMEM),
            pl.BlockSpec((rb, hp), lambda i: (jnp.minimum(i, g - 1), i * 0)),
            pl.BlockSpec((n, hp), lambda i: (i * 0, i * 0)),
        ],
        out_specs=[
            pl.BlockSpec((rb, _LANES), lambda i: (jnp.maximum(i - 1, 0), i * 0)),
            pl.BlockSpec((rb, _LANES), lambda i: (jnp.maximum(i - 1, 0), i * 0)),
        ],
        out_shape=[
            jax.ShapeDtypeStruct((n, _LANES), jnp.float32),
            jax.ShapeDtypeStruct((n, _LANES), jnp.int32),
        ],
        scratch_shapes=[pltpu.VMEM((2, rb, n), jnp.float32)],
    )(alpha, e, e)

    fa = fixed_edge_attr.reshape(e_num // _LANES, _LANES)
    fa_scaled = pl.pallas_call(
        _scale_body,
        grid=(1,),
        in_specs=[
            pl.BlockSpec((1,), lambda i: (i * 0,), memory_space=pltpu.SMEM),
            pl.BlockSpec(fa.shape, lambda i: (i * 0, i * 0)),
        ],
        out_specs=pl.BlockSpec(fa.shape, lambda i: (i * 0, i * 0)),
        out_shape=jax.ShapeDtypeStruct(fa.shape, jnp.float32),
    )(alpha, fa)

    src = jnp.repeat(jnp.arange(n, dtype=idx_dtype), _K)
    dst = idx[:, :_K].reshape(-1).astype(idx_dtype)
    dyn_edge_index = jnp.stack([src, dst], axis=0)
    combined_edge_index = jnp.concatenate([fixed_edge_index, dyn_edge_index], axis=1)
    combined_edge_attr = jnp.concatenate(
        [fa_scaled.reshape(e_num, 1), vals[:, :_K].reshape(-1, 1)], axis=0
    ).astype(attr_dtype)
    return combined_edge_index, combined_edge_attr
